# baseline (speedup 1.0000x reference)
import jax
import jax.numpy as jnp
from jax.experimental import pallas as pl
from jax.experimental.pallas import tpu as pltpu

ROWS = 8192
DIM = 4096
BM = 256
BK = 512
M_TILES = ROWS // BM
K_TILES = DIM // BK


def _body(p_ref, x_ref, o_ref, pmat_ref):
    m = pl.program_id(0)
    k = pl.program_id(1)

    @pl.when(m == 0)
    def _build_pmat():
        s = jax.lax.broadcasted_iota(jnp.int32, (BK, DIM), 0) + k * BK
        permb = jnp.broadcast_to(p_ref[0][None, :], (BK, DIM))
        pmat_ref[k] = (s == permb).astype(jnp.bfloat16)

    xb = x_ref[...].astype(jnp.bfloat16)
    prod = jnp.dot(xb, pmat_ref[k], preferred_element_type=jnp.float32)

    @pl.when(k == 0)
    def _init():
        o_ref[...] = prod

    @pl.when(k != 0)
    def _acc():
        o_ref[...] += prod


def kernel(x, perm):
    perm2d = perm.reshape(1, DIM)
    return pl.pallas_call(
        _body,
        grid=(M_TILES, K_TILES),
        in_specs=[
            pl.BlockSpec((1, DIM), lambda m, k: (0, 0)),
            pl.BlockSpec((BM, BK), lambda m, k: (m, k)),
        ],
        out_specs=pl.BlockSpec((BM, DIM), lambda m, k: (m, 0)),
        out_shape=jax.ShapeDtypeStruct((ROWS, DIM), x.dtype),
        scratch_shapes=[pltpu.VMEM((K_TILES, BK, DIM), jnp.bfloat16)],
    )(perm2d, x)


# speedup vs baseline: 1.5804x; 1.5804x over previous
import jax
import jax.numpy as jnp
from jax.experimental import pallas as pl
from jax.experimental.pallas import tpu as pltpu

ROWS = 8192
DIM = 4096
BM = 256
M_TILES = ROWS // BM


def _body(p_ref, x_ref, o_ref, pmat_ref):
    m = pl.program_id(0)

    @pl.when(m == 0)
    def _build_pmat():
        s = jax.lax.broadcasted_iota(jnp.int32, (DIM, DIM), 0)
        permb = jnp.broadcast_to(p_ref[0][None, :], (DIM, DIM))
        pmat_ref[...] = (s == permb).astype(jnp.bfloat16)

    xb = x_ref[...].astype(jnp.bfloat16)
    o_ref[...] = jnp.dot(xb, pmat_ref[...], preferred_element_type=jnp.float32)


def kernel(x, perm):
    perm2d = perm.reshape(1, DIM)
    return pl.pallas_call(
        _body,
        grid=(M_TILES,),
        in_specs=[
            pl.BlockSpec((1, DIM), lambda m: (0, 0)),
            pl.BlockSpec((BM, DIM), lambda m: (m, 0)),
        ],
        out_specs=pl.BlockSpec((BM, DIM), lambda m: (m, 0)),
        out_shape=jax.ShapeDtypeStruct((ROWS, DIM), x.dtype),
        scratch_shapes=[pltpu.VMEM((DIM, DIM), jnp.bfloat16)],
    )(perm2d, x)
